# baseline (speedup 1.0000x reference)
import jax
import jax.numpy as jnp
from jax.experimental import pallas as pl
from jax.experimental.pallas import tpu as pltpu

_H = 512
_W = 512
_K = 5
_THR = 0.3
_SR = 64


def _nms_topk_kernel(hm_ref, vals_ref, idx_ref, p_ref):
    for img in range(hm_ref.shape[0]):
        _one_image(hm_ref, vals_ref, idx_ref, p_ref, img)


def _one_image(hm_ref, vals_ref, idx_ref, p_ref, img):
    for s in range(_H // _SR):
        r0 = s * _SR
        mid = hm_ref[img, 0, pl.ds(r0, _SR), :]
        if s == 0:
            up = jnp.concatenate(
                [hm_ref[img, 0, 0:1, :], hm_ref[img, 0, 0:_SR - 1, :]],
                axis=0)
        else:
            up = hm_ref[img, 0, pl.ds(r0 - 1, _SR), :]
        if s == _H // _SR - 1:
            dn = jnp.concatenate(
                [hm_ref[img, 0, r0 + 1:_H, :], hm_ref[img, 0, _H - 1:_H, :]],
                axis=0)
        else:
            dn = hm_ref[img, 0, pl.ds(r0 + 1, _SR), :]
        m = jnp.maximum(mid, jnp.maximum(up, dn))
        lane_sw = jax.lax.broadcasted_iota(jnp.int32, (_SR, _W), 1)
        lf = jnp.where(lane_sw == _W - 1, m, pltpu.roll(m, _W - 1, 1))
        rt = jnp.where(lane_sw == 0, m, pltpu.roll(m, 1, 1))
        pooled = jnp.maximum(m, jnp.maximum(lf, rt))
        p_ref[img, pl.ds(r0, _SR), :] = jnp.where(pooled == mid, mid,
                                                  jnp.float32(0.0))

    _HQ = _H // 4
    s0 = p_ref[img, 0:_HQ, :]
    s1 = p_ref[img, _HQ:2 * _HQ, :]
    s2 = p_ref[img, 2 * _HQ:3 * _HQ, :]
    s3 = p_ref[img, 3 * _HQ:, :]
    ba = s1 > s0
    a = jnp.maximum(s0, s1)
    bb = s3 > s2
    b = jnp.maximum(s2, s3)
    takeb = b > a
    q = jnp.maximum(a, b)
    min_ab = jnp.minimum(a, b)
    la = jnp.where(ba, s0, s1)
    lb = jnp.where(bb, s2, s3)
    lw = jnp.where(takeb, lb, la)
    sec4 = jnp.maximum(min_ab, lw)
    rh = jax.lax.broadcasted_iota(jnp.int32, (_HQ, _W), 0)
    ja = ba.astype(jnp.int32)
    jb = bb.astype(jnp.int32) + 2
    jsel = jnp.where(takeb, jb, ja)
    rowfull = rh + jsel * _HQ

    colmax = jnp.max(q, axis=0, keepdims=True)
    colrow = jnp.min(jnp.where(q == colmax, rowfull, _H),
                     axis=0, keepdims=True)

    lane_w = jax.lax.broadcasted_iota(jnp.int32, (1, _W), 1)
    big = jnp.int32(_H * _W)
    flat = colrow * _W + lane_w

    avT = jnp.transpose(jnp.broadcast_to(colmax, (8, _W)))[:, :1]
    flatT = jnp.transpose(jnp.broadcast_to(flat, (8, _W)))[:, :1]
    better = (avT > colmax) | ((avT == colmax) & (flatT < flat))
    rank = jnp.sum(better.astype(jnp.int32), axis=0, keepdims=True)

    sub8 = jax.lax.broadcasted_iota(jnp.int32, (8, _W), 0)
    maskk = sub8 == rank
    vals8 = jnp.max(jnp.where(maskk, colmax, jnp.float32(-1.0)),
                    axis=1, keepdims=True)
    idx8 = jnp.min(jnp.where(maskk, flat, big), axis=1, keepdims=True)

    taken = rank < jnp.int32(_K)
    rep = (q == colmax) & (rowfull == colrow)
    remv = jnp.where(rep, sec4, q)
    sec = jnp.max(jnp.where(taken, remv, jnp.float32(-1.0)))
    v5 = jnp.min(jnp.where(taken, colmax, jnp.float32(jnp.inf)))
    ok = sec < v5

    def _fast(_):
        return vals8, idx8

    def _slow(_):
        rowiota = jax.lax.broadcasted_iota(jnp.int32, (_H, _W), 0)
        flatiota = rowiota * _W + jax.lax.broadcasted_iota(
            jnp.int32, (_H, _W), 1)
        sub81 = jax.lax.broadcasted_iota(jnp.int32, (8, 1), 0)
        pp = p_ref[img]
        vv = jnp.zeros((8, 1), jnp.float32)
        iv = jnp.zeros((8, 1), jnp.int32)
        for k in range(_K):
            v2 = jnp.max(pp)
            f2 = jnp.min(jnp.where(pp == v2, flatiota, big))
            vv = jnp.where(sub81 == k, v2, vv)
            iv = jnp.where(sub81 == k, f2, iv)
            if k < _K - 1:
                pp = jnp.where(flatiota == f2, jnp.float32(-1.0), pp)
        return vv, iv

    vr, ir = jax.lax.cond(ok, _fast, _slow, None)
    vals_ref[img, :, 0:1] = vr
    idx_ref[img, :, 0:1] = ir


@jax.jit
def kernel(heatmap):
    B = heatmap.shape[0]
    ipp = 2 if B % 2 == 0 else 1
    vals, idx = pl.pallas_call(
        _nms_topk_kernel,
        grid=(B // ipp,),
        in_specs=[pl.BlockSpec((ipp, 1, _H, _W), lambda b: (b, 0, 0, 0))],
        out_specs=[
            pl.BlockSpec((ipp, 8, 128), lambda b: (b, 0, 0)),
            pl.BlockSpec((ipp, 8, 128), lambda b: (b, 0, 0)),
        ],
        out_shape=[
            jax.ShapeDtypeStruct((B, 8, 128), jnp.float32),
            jax.ShapeDtypeStruct((B, 8, 128), jnp.int32),
        ],
        scratch_shapes=[pltpu.VMEM((ipp, _H, _W), jnp.float32)],
        compiler_params=pltpu.CompilerParams(
            dimension_semantics=("parallel",)),
    )(heatmap)
    top_vals = vals[:, :_K, 0]
    top_idx = idx[:, :_K, 0]
    valid_mask = top_vals >= _THR
    row_idx = (top_idx // _W).astype(jnp.float32)
    col_idx = (top_idx % _W).astype(jnp.float32)
    norm_y = 2.0 * row_idx / float(_H - 1) - 1.0
    norm_x = 2.0 * col_idx / float(_W - 1) - 1.0
    centers = jnp.stack([norm_x, norm_y], axis=-1)
    centers = centers * valid_mask[..., None].astype(jnp.float32)
    return (centers, valid_mask, top_vals)


# speedup vs baseline: 1.0530x; 1.0530x over previous
import jax
import jax.numpy as jnp
from jax.experimental import pallas as pl
from jax.experimental.pallas import tpu as pltpu

_H = 512
_W = 512
_K = 5
_THR = 0.3
_SR = 64


def _nms_topk_kernel(hm_ref, vals_ref, idx_ref, p_ref):
    for s in range(_H // _SR):
        r0 = s * _SR
        mid = hm_ref[0, 0, pl.ds(r0, _SR), :]
        if s == 0:
            up = jnp.concatenate(
                [hm_ref[0, 0, 0:1, :], hm_ref[0, 0, 0:_SR - 1, :]], axis=0)
        else:
            up = hm_ref[0, 0, pl.ds(r0 - 1, _SR), :]
        if s == _H // _SR - 1:
            dn = jnp.concatenate(
                [hm_ref[0, 0, r0 + 1:_H, :], hm_ref[0, 0, _H - 1:_H, :]],
                axis=0)
        else:
            dn = hm_ref[0, 0, pl.ds(r0 + 1, _SR), :]
        m = jnp.maximum(mid, jnp.maximum(up, dn))
        lane_sw = jax.lax.broadcasted_iota(jnp.int32, (_SR, _W), 1)
        lf = jnp.where(lane_sw == _W - 1, m, pltpu.roll(m, _W - 1, 1))
        rt = jnp.where(lane_sw == 0, m, pltpu.roll(m, 1, 1))
        pooled = jnp.maximum(m, jnp.maximum(lf, rt))
        p_ref[pl.ds(r0, _SR), :] = jnp.where(pooled == mid, mid,
                                             jnp.float32(0.0))

    _HQ = _H // 4
    s0 = p_ref[0:_HQ, :]
    s1 = p_ref[_HQ:2 * _HQ, :]
    s2 = p_ref[2 * _HQ:3 * _HQ, :]
    s3 = p_ref[3 * _HQ:, :]
    ba = s1 > s0
    a = jnp.maximum(s0, s1)
    bb = s3 > s2
    b = jnp.maximum(s2, s3)
    takeb = b > a
    q = jnp.maximum(a, b)
    min_ab = jnp.minimum(a, b)
    la = jnp.where(ba, s0, s1)
    lb = jnp.where(bb, s2, s3)
    lw = jnp.where(takeb, lb, la)
    sec4 = jnp.maximum(min_ab, lw)
    rh = jax.lax.broadcasted_iota(jnp.int32, (_HQ, _W), 0)
    ja = ba.astype(jnp.int32)
    jb = bb.astype(jnp.int32) + 2
    jsel = jnp.where(takeb, jb, ja)
    rowfull = rh + jsel * _HQ

    colmax = jnp.max(q, axis=0, keepdims=True)
    colrow = jnp.min(jnp.where(q == colmax, rowfull, _H),
                     axis=0, keepdims=True)

    lane_w = jax.lax.broadcasted_iota(jnp.int32, (1, _W), 1)
    big = jnp.int32(_H * _W)
    flat = colrow * _W + lane_w

    avT = jnp.transpose(jnp.broadcast_to(colmax, (8, _W)))[:, :1]
    better = avT > colmax
    rank = jnp.sum(better.astype(jnp.int32), axis=0, keepdims=True)

    sub8 = jax.lax.broadcasted_iota(jnp.int32, (8, _W), 0)
    maskk = sub8 == rank
    vals8 = jnp.max(jnp.where(maskk, colmax, jnp.float32(-1.0)),
                    axis=1, keepdims=True)
    idx8 = jnp.min(jnp.where(maskk, flat, big), axis=1, keepdims=True)
    filled = jnp.min(vals8[:_K, :]) >= jnp.float32(0.0)

    taken = rank < jnp.int32(_K)
    rep = (q == colmax) & (rowfull == colrow)
    remv = jnp.where(rep, sec4, q)
    sec = jnp.max(jnp.where(taken, remv, jnp.float32(-1.0)))
    v5 = jnp.min(jnp.where(taken, colmax, jnp.float32(jnp.inf)))
    ok = (sec < v5) & filled

    def _fast(_):
        return vals8, idx8

    def _slow(_):
        rowiota = jax.lax.broadcasted_iota(jnp.int32, (_H, _W), 0)
        flatiota = rowiota * _W + jax.lax.broadcasted_iota(
            jnp.int32, (_H, _W), 1)
        sub81 = jax.lax.broadcasted_iota(jnp.int32, (8, 1), 0)
        pp = p_ref[...]
        vv = jnp.zeros((8, 1), jnp.float32)
        iv = jnp.zeros((8, 1), jnp.int32)
        for k in range(_K):
            v2 = jnp.max(pp)
            f2 = jnp.min(jnp.where(pp == v2, flatiota, big))
            vv = jnp.where(sub81 == k, v2, vv)
            iv = jnp.where(sub81 == k, f2, iv)
            if k < _K - 1:
                pp = jnp.where(flatiota == f2, jnp.float32(-1.0), pp)
        return vv, iv

    vr, ir = jax.lax.cond(ok, _fast, _slow, None)
    vals_ref[0, :, 0:1] = vr
    idx_ref[0, :, 0:1] = ir


@jax.jit
def kernel(heatmap):
    B = heatmap.shape[0]
    vals, idx = pl.pallas_call(
        _nms_topk_kernel,
        grid=(B,),
        in_specs=[pl.BlockSpec((1, 1, _H, _W), lambda b: (b, 0, 0, 0))],
        out_specs=[
            pl.BlockSpec((1, 8, 128), lambda b: (b, 0, 0)),
            pl.BlockSpec((1, 8, 128), lambda b: (b, 0, 0)),
        ],
        out_shape=[
            jax.ShapeDtypeStruct((B, 8, 128), jnp.float32),
            jax.ShapeDtypeStruct((B, 8, 128), jnp.int32),
        ],
        scratch_shapes=[pltpu.VMEM((_H, _W), jnp.float32)],
        compiler_params=pltpu.CompilerParams(
            dimension_semantics=("parallel",)),
    )(heatmap)
    top_vals = vals[:, :_K, 0]
    top_idx = idx[:, :_K, 0]
    valid_mask = top_vals >= _THR
    row_idx = (top_idx // _W).astype(jnp.float32)
    col_idx = (top_idx % _W).astype(jnp.float32)
    norm_y = 2.0 * row_idx / float(_H - 1) - 1.0
    norm_x = 2.0 * col_idx / float(_W - 1) - 1.0
    centers = jnp.stack([norm_x, norm_y], axis=-1)
    centers = centers * valid_mask[..., None].astype(jnp.float32)
    return (centers, valid_mask, top_vals)
